# initial kernel scaffold (unmeasured)
import jax
import jax.numpy as jnp
from jax import lax
from jax.experimental import pallas as pl
from jax.experimental.pallas import tpu as pltpu

N_DEV = 16


def kernel(x, w_mat):
    m_total, k_per = x.shape
    k_total, n_out = w_mat.shape
    m_per = m_total // N_DEV

    def body(x_ref, w_ref, out_ref, recv_buf, w_buf, send_sems, recv_sems, w_sem):
        my_i = lax.axis_index("i")

        barrier_sem = pltpu.get_barrier_semaphore()
        for dj in range(1, N_DEV):
            peer = (my_i + dj) % N_DEV
            pl.semaphore_signal(
                barrier_sem, inc=1,
                device_id=(peer,), device_id_type=pl.DeviceIdType.MESH,
            )
        pl.semaphore_wait(barrier_sem, N_DEV - 1)

        rdmas = [None] * N_DEV
        for dj in range(1, N_DEV):
            dst = (my_i + dj) % N_DEV
            rdma = pltpu.make_async_remote_copy(
                src_ref=x_ref.at[pl.ds(dst * m_per, m_per), :],
                dst_ref=recv_buf.at[dj],
                send_sem=send_sems.at[dj],
                recv_sem=recv_sems.at[dj],
                device_id=(dst,),
                device_id_type=pl.DeviceIdType.MESH,
            )
            rdma.start()
            rdmas[dj] = rdma

        for k in range(N_DEV):
            j = (my_i - k) % N_DEV
            w_dma = pltpu.make_async_copy(
                w_ref.at[pl.ds(j * k_per, k_per), :],
                w_buf,
                w_sem,
            )
            w_dma.start()
            w_dma.wait()
            if k == 0:
                a = x_ref[pl.ds(my_i * m_per, m_per), :]
            else:
                rdmas[k].wait_recv()
                a = recv_buf[k]
            acc = jnp.dot(a, w_buf[...], preferred_element_type=jnp.float32)
            if k == 0:
                out_ref[...] = acc
            else:
                out_ref[...] += acc

        y = out_ref[...]
        c = 0.7978845608028654
        out_ref[...] = 0.5 * y * (1.0 + jnp.tanh(c * (y + 0.044715 * y * y * y)))

        for dj in range(1, N_DEV):
            rdmas[dj].wait_send()

    return pl.pallas_call(
        body,
        out_shape=jax.ShapeDtypeStruct((m_per, n_out), jnp.float32),
        in_specs=[
            pl.BlockSpec(memory_space=pltpu.VMEM),
            pl.BlockSpec(memory_space=pltpu.ANY),
        ],
        out_specs=pl.BlockSpec(memory_space=pltpu.VMEM),
        scratch_shapes=[
            pltpu.VMEM((N_DEV, m_per, k_per), jnp.float32),
            pltpu.VMEM((k_per, n_out), jnp.float32),
            pltpu.SemaphoreType.DMA((N_DEV,)),
            pltpu.SemaphoreType.DMA((N_DEV,)),
            pltpu.SemaphoreType.DMA,
        ],
        compiler_params=pltpu.CompilerParams(collective_id=0),
    )(x, w_mat)


# baseline (device time: 92039 ns/iter reference)
import jax
import jax.numpy as jnp
from jax import lax
from jax.experimental import pallas as pl
from jax.experimental.pallas import tpu as pltpu

N_DEV = 16


def kernel(x, w_mat):
    m_total, k_per = x.shape
    k_total, n_out = w_mat.shape
    m_per = m_total // N_DEV

    def body(x_ref, w_ref, out_ref, recv_buf, w_buf, send_sems, recv_sems, w_sem):
        my_i = lax.axis_index("i")

        barrier_sem = pltpu.get_barrier_semaphore()
        for dj in range(1, N_DEV):
            peer = (my_i + dj) % N_DEV
            pl.semaphore_signal(
                barrier_sem, inc=1,
                device_id=(peer,), device_id_type=pl.DeviceIdType.MESH,
            )
        pl.semaphore_wait(barrier_sem, N_DEV - 1)

        rdmas = [None] * N_DEV
        for dj in range(1, N_DEV):
            dst = (my_i + dj) % N_DEV
            rdma = pltpu.make_async_remote_copy(
                src_ref=x_ref.at[pl.ds(dst * m_per, m_per), :],
                dst_ref=recv_buf.at[dj],
                send_sem=send_sems.at[dj],
                recv_sem=recv_sems.at[dj],
                device_id=(dst,),
                device_id_type=pl.DeviceIdType.MESH,
            )
            rdma.start()
            rdmas[dj] = rdma

        for k in range(N_DEV):
            j = (my_i - k) % N_DEV
            w_dma = pltpu.make_async_copy(
                w_ref.at[pl.ds(j * k_per, k_per), :],
                w_buf,
                w_sem,
            )
            w_dma.start()
            w_dma.wait()
            if k == 0:
                a = x_ref[pl.ds(my_i * m_per, m_per), :]
            else:
                rdmas[k].wait_recv()
                a = recv_buf[k]
            acc = jnp.dot(a, w_buf[...], preferred_element_type=jnp.float32)
            if k == 0:
                out_ref[...] = acc
            else:
                out_ref[...] += acc

        y = out_ref[...]
        c = 0.7978845608028654
        out_ref[...] = 0.5 * y * (1.0 + jnp.tanh(c * (y + 0.044715 * y * y * y)))

        for dj in range(1, N_DEV):
            rdmas[dj].wait_send()

    return pl.pallas_call(
        body,
        out_shape=jax.ShapeDtypeStruct((m_per, n_out), jnp.float32),
        in_specs=[
            pl.BlockSpec(memory_space=pltpu.VMEM),
            pl.BlockSpec(memory_space=pl.ANY),
        ],
        out_specs=pl.BlockSpec(memory_space=pltpu.VMEM),
        scratch_shapes=[
            pltpu.VMEM((N_DEV, m_per, k_per), jnp.float32),
            pltpu.VMEM((k_per, n_out), jnp.float32),
            pltpu.SemaphoreType.DMA((N_DEV,)),
            pltpu.SemaphoreType.DMA((N_DEV,)),
            pltpu.SemaphoreType.DMA,
        ],
        compiler_params=pltpu.CompilerParams(collective_id=0),
    )(x, w_mat)


# device time: 70083 ns/iter; 1.3133x vs baseline; 1.3133x over previous
import jax
import jax.numpy as jnp
from jax import lax
from jax.experimental import pallas as pl
from jax.experimental.pallas import tpu as pltpu

N_DEV = 16


def kernel(x, w_mat):
    m_total, k_per = x.shape
    k_total, n_out = w_mat.shape
    m_per = m_total // N_DEV

    def body(x_ref, w_ref, out_ref, recv_buf, w_buf, send_sems, recv_sems, w_sems):
        my_i = lax.axis_index("i")

        def start_w(k):
            j = (my_i - k) % N_DEV
            dma = pltpu.make_async_copy(
                w_ref.at[pl.ds(j * k_per, k_per), :],
                w_buf.at[k % 2],
                w_sems.at[k % 2],
            )
            dma.start()
            return dma

        w_dmas = [None] * N_DEV
        w_dmas[0] = start_w(0)
        w_dmas[1] = start_w(1)

        barrier_sem = pltpu.get_barrier_semaphore()
        for dj in range(1, N_DEV):
            peer = (my_i + dj) % N_DEV
            pl.semaphore_signal(
                barrier_sem, inc=1,
                device_id=(peer,), device_id_type=pl.DeviceIdType.MESH,
            )
        pl.semaphore_wait(barrier_sem, N_DEV - 1)

        rdmas = [None] * N_DEV
        for dj in range(1, N_DEV):
            dst = (my_i + dj) % N_DEV
            rdma = pltpu.make_async_remote_copy(
                src_ref=x_ref.at[pl.ds(dst * m_per, m_per), :],
                dst_ref=recv_buf.at[dj],
                send_sem=send_sems.at[dj],
                recv_sem=recv_sems.at[dj],
                device_id=(dst,),
                device_id_type=pl.DeviceIdType.MESH,
            )
            rdma.start()
            rdmas[dj] = rdma

        for k in range(N_DEV):
            w_dmas[k].wait()
            if k == 0:
                a = x_ref[pl.ds(my_i * m_per, m_per), :]
            else:
                rdmas[k].wait_recv()
                a = recv_buf[k]
            a16 = a.astype(jnp.bfloat16)
            w16 = w_buf[k % 2].astype(jnp.bfloat16)
            acc = jnp.dot(a16, w16, preferred_element_type=jnp.float32)
            if k == 0:
                out_ref[...] = acc
            else:
                out_ref[...] += acc
            if k + 2 < N_DEV:
                w_dmas[k + 2] = start_w(k + 2)

        y = out_ref[...]
        c = 0.7978845608028654
        out_ref[...] = 0.5 * y * (1.0 + jnp.tanh(c * (y + 0.044715 * y * y * y)))

        for dj in range(1, N_DEV):
            rdmas[dj].wait_send()

    return pl.pallas_call(
        body,
        out_shape=jax.ShapeDtypeStruct((m_per, n_out), jnp.float32),
        in_specs=[
            pl.BlockSpec(memory_space=pltpu.VMEM),
            pl.BlockSpec(memory_space=pl.ANY),
        ],
        out_specs=pl.BlockSpec(memory_space=pltpu.VMEM),
        scratch_shapes=[
            pltpu.VMEM((N_DEV, m_per, k_per), jnp.float32),
            pltpu.VMEM((2, k_per, n_out), jnp.float32),
            pltpu.SemaphoreType.DMA((N_DEV,)),
            pltpu.SemaphoreType.DMA((N_DEV,)),
            pltpu.SemaphoreType.DMA((2,)),
        ],
        compiler_params=pltpu.CompilerParams(collective_id=0),
    )(x, w_mat)
